# baseline (speedup 1.0000x reference)
import functools

import jax
import jax.numpy as jnp
from jax import lax
from jax.experimental import pallas as pl
from jax.experimental.pallas import tpu as pltpu
from jax.experimental.pallas import tpu_sc as plsc

PW = 0.5
BN, C, HP = 48, 2, 64
B, H = 2, 160
NPB = BN // B
PLANES = B * C * H
NW = 32
PPW = PLANES // NW
L = 16
NPLB = 3
NPAB = 3
PR = 16
NSUB = HP // PR
RU = 4
SU = 2
KP = HP // L
KH = H // L


def _sc_body(patches_hbm, vol_hbm, off_hbm, out_hbm, plane_v, patch_v, off_t,
             off_s, wl_s, load_sem, store_sem, patch_sem):
    wid = lax.axis_index("s") * 2 + lax.axis_index("c")
    pltpu.sync_copy(off_hbm, off_t)
    lane = lax.iota(jnp.int32, L)

    def extract_body(i, carry):
        for k in range(3):
            ii = jnp.full((L,), i * 3 + k, jnp.int32)
            v = plsc.load_gather(off_t, [ii])
            off_s[i * 3 + k] = jnp.max(v)
        return carry

    lax.fori_loop(0, BN, extract_body, 0)

    def decode(t):
        p = t * NW + wid
        return p // (C * H), (p // H) % C, p % H

    def issue_load(t):
        b, c, h = decode(t)
        pltpu.async_copy(vol_hbm.at[b, c, h], plane_v.at[t % NPLB],
                         load_sem.at[t % NPLB])

    issue_load(0)
    issue_load(1)

    def iter_body(t, carry):
        buf = lax.rem(t, NPLB)
        b, c, h = decode(t)

        def wl_body(j, m):
            i = b * NPB + j
            dh = h - off_s[i * 3]
            cond = (dh >= 0) & (dh < HP)

            @pl.when(cond)
            def _():
                wl_s[m * 2] = i
                wl_s[m * 2 + 1] = dh

            return m + cond.astype(jnp.int32)

        m = lax.fori_loop(0, NPB, wl_body, 0)

        def issue_sub(u):
            j = u // NSUB
            sub = lax.rem(u, NSUB)
            pltpu.async_copy(
                patches_hbm.at[wl_s[j * 2], c, wl_s[j * 2 + 1],
                               pl.ds(sub * PR, PR)],
                patch_v.at[lax.rem(u, NPAB)],
                patch_sem.at[lax.rem(u, NPAB)])

        nu = m * NSUB
        for u0 in range(NPAB - 1):
            @pl.when(u0 < nu)
            def _():
                issue_sub(u0)

        pltpu.make_async_copy(vol_hbm.at[b, c, h], plane_v.at[buf],
                              load_sem.at[buf]).wait()

        def sub_body(u, carry):
            pb = lax.rem(u, NPAB)
            j = u // NSUB
            sub = lax.rem(u, NSUB)
            i = wl_s[j * 2]

            @pl.when(u + NPAB - 1 < nu)
            def _():
                issue_sub(u + NPAB - 1)

            dh = wl_s[j * 2 + 1]
            pltpu.make_async_copy(
                patches_hbm.at[i, c, dh, pl.ds(sub * PR, PR)],
                patch_v.at[pb], patch_sem.at[pb]).wait()

            s1 = off_s[i * 3 + 1]
            s2 = off_s[i * 3 + 2]
            cols = tuple(lane + (s2 + k * L) for k in range(KP))
            row0 = jnp.full((L,), s1 + sub * PR, jnp.int32)

            def row_body(q, row_vec):
                r = q * RU
                xs = [patch_v[pb, r + rr, pl.ds(k * L, L)]
                      for rr in range(RU) for k in range(KP)]
                for rr in range(RU):
                    rv = row_vec + rr if rr else row_vec
                    for k in range(KP):
                        plsc.addupdate_scatter(plane_v.at[buf],
                                               [rv, cols[k]],
                                               xs[rr * KP + k])
                return row_vec + RU

            lax.fori_loop(0, PR // RU, row_body, row0)
            return carry

        lax.fori_loop(0, nu, sub_body, 0)

        def scale_body(q, cc):
            r = q * SU
            xs = [plane_v[buf, r + rr, pl.ds(k * L, L)] * PW
                  for rr in range(SU) for k in range(KH)]
            for rr in range(SU):
                for k in range(KH):
                    plane_v[buf, r + rr, pl.ds(k * L, L)] = xs[rr * KH + k]
            return cc

        lax.fori_loop(0, H // SU, scale_body, 0)
        pltpu.async_copy(plane_v.at[buf], out_hbm.at[b, c, h],
                         store_sem.at[buf])

        @pl.when(t + 2 < PPW)
        def _():
            nbuf = lax.rem(t + 2, NPLB)

            @pl.when(t >= 1)
            def _():
                bp, cp, hp_ = decode(t - 1)
                pltpu.make_async_copy(plane_v.at[nbuf],
                                      out_hbm.at[bp, cp, hp_],
                                      store_sem.at[nbuf]).wait()

            issue_load(t + 2)

        return carry

    lax.fori_loop(0, PPW, iter_body, 0)

    for t in range(PPW - NPLB, PPW):
        b, c, h = decode(t)
        pltpu.make_async_copy(plane_v.at[t % NPLB], out_hbm.at[b, c, h],
                              store_sem.at[t % NPLB]).wait()


@jax.jit
def kernel(patches, vol, offsets):
    mesh = plsc.VectorSubcoreMesh(core_axis_name="c", subcore_axis_name="s")
    run = pl.kernel(
        _sc_body,
        out_type=jax.ShapeDtypeStruct((B, C, H, H, H), jnp.float32),
        mesh=mesh,
        scratch_types=[
            pltpu.VMEM((NPLB, H, H), jnp.float32),
            pltpu.VMEM((NPAB, PR, HP), jnp.float32),
            pltpu.VMEM((BN * 3,), jnp.int32),
            pltpu.SMEM((BN * 3,), jnp.int32),
            pltpu.SMEM((NPB * 2,), jnp.int32),
            pltpu.SemaphoreType.DMA((NPLB,)),
            pltpu.SemaphoreType.DMA((NPLB,)),
            pltpu.SemaphoreType.DMA((NPAB,)),
        ],
        compiler_params=pltpu.CompilerParams(
            use_tc_tiling_on_sc=True, needs_layout_passes=False),
    )
    return run(patches, vol, offsets.reshape(-1))


# speedup vs baseline: 1.4457x; 1.4457x over previous
import functools

import jax
import jax.numpy as jnp
from jax import lax
from jax.experimental import pallas as pl
from jax.experimental.pallas import tpu as pltpu
from jax.experimental.pallas import tpu_sc as plsc

PW = 0.5
BN, C, HP = 48, 2, 64
B, H = 2, 160
NPB = BN // B
PLANES = B * C * H
NW = 32
PPW = PLANES // NW
L = 16
NPLB = 2
NPAB = 3
PR = 64
NSUB = HP // PR
RU = 4
SU = 2
KP = HP // L
KH = H // L


def _sc_body(patches_hbm, vol_hbm, off_hbm, out_hbm, plane_v, patch_v, off_t,
             off_s, wl_s, load_sem, store_sem, patch_sem):
    wid = lax.axis_index("s") * 2 + lax.axis_index("c")
    pltpu.sync_copy(off_hbm, off_t)
    lane = lax.iota(jnp.int32, L)

    def extract_body(i, carry):
        for k in range(3):
            ii = jnp.full((L,), i * 3 + k, jnp.int32)
            v = plsc.load_gather(off_t, [ii])
            off_s[i * 3 + k] = jnp.max(v)
        return carry

    lax.fori_loop(0, BN, extract_body, 0)

    def decode(t):
        p = t * NW + wid
        return p // (C * H), (p // H) % C, p % H

    def issue_load(t):
        b, c, h = decode(t)
        pltpu.async_copy(vol_hbm.at[b, c, h], plane_v.at[t % NPLB],
                         load_sem.at[t % NPLB])

    for t0 in range(NPLB - 1):
        issue_load(t0)

    def iter_body(t, carry):
        buf = lax.rem(t, NPLB)
        b, c, h = decode(t)

        def wl_body(j, m):
            i = b * NPB + j
            dh = h - off_s[i * 3]
            cond = (dh >= 0) & (dh < HP)

            @pl.when(cond)
            def _():
                wl_s[m * 2] = i
                wl_s[m * 2 + 1] = dh

            return m + cond.astype(jnp.int32)

        m = lax.fori_loop(0, NPB, wl_body, 0)

        def issue_sub(u):
            j = u // NSUB
            sub = lax.rem(u, NSUB)
            pltpu.async_copy(
                patches_hbm.at[wl_s[j * 2], c, wl_s[j * 2 + 1],
                               pl.ds(sub * PR, PR)],
                patch_v.at[lax.rem(u, NPAB)],
                patch_sem.at[lax.rem(u, NPAB)])

        nu = m * NSUB
        for u0 in range(NPAB - 1):
            @pl.when(u0 < nu)
            def _():
                issue_sub(u0)

        pltpu.make_async_copy(vol_hbm.at[b, c, h], plane_v.at[buf],
                              load_sem.at[buf]).wait()

        def sub_body(u, carry):
            pb = lax.rem(u, NPAB)
            j = u // NSUB
            sub = lax.rem(u, NSUB)
            i = wl_s[j * 2]

            @pl.when(u + NPAB - 1 < nu)
            def _():
                issue_sub(u + NPAB - 1)

            dh = wl_s[j * 2 + 1]
            pltpu.make_async_copy(
                patches_hbm.at[i, c, dh, pl.ds(sub * PR, PR)],
                patch_v.at[pb], patch_sem.at[pb]).wait()

            s1 = off_s[i * 3 + 1]
            s2 = off_s[i * 3 + 2]
            cols = tuple(lane + (s2 + k * L) for k in range(KP))
            row0 = jnp.full((L,), s1 + sub * PR, jnp.int32)

            def row_body(q, row_vec):
                r = q * RU
                xs = [patch_v[pb, r + rr, pl.ds(k * L, L)]
                      for rr in range(RU) for k in range(KP)]
                for rr in range(RU):
                    rv = row_vec + rr if rr else row_vec
                    for k in range(KP):
                        plsc.addupdate_scatter(plane_v.at[buf],
                                               [rv, cols[k]],
                                               xs[rr * KP + k])
                return row_vec + RU

            lax.fori_loop(0, PR // RU, row_body, row0)
            return carry

        lax.fori_loop(0, nu, sub_body, 0)

        def scale_body(q, cc):
            r = q * SU
            xs = [plane_v[buf, r + rr, pl.ds(k * L, L)] * PW
                  for rr in range(SU) for k in range(KH)]
            for rr in range(SU):
                for k in range(KH):
                    plane_v[buf, r + rr, pl.ds(k * L, L)] = xs[rr * KH + k]
            return cc

        lax.fori_loop(0, H // SU, scale_body, 0)
        pltpu.async_copy(plane_v.at[buf], out_hbm.at[b, c, h],
                         store_sem.at[buf])

        @pl.when(t + NPLB - 1 < PPW)
        def _():
            nbuf = lax.rem(t + NPLB - 1, NPLB)

            @pl.when(t >= 1)
            def _():
                bp, cp, hp_ = decode(t - 1)
                pltpu.make_async_copy(plane_v.at[nbuf],
                                      out_hbm.at[bp, cp, hp_],
                                      store_sem.at[nbuf]).wait()

            issue_load(t + NPLB - 1)

        return carry

    lax.fori_loop(0, PPW, iter_body, 0)

    for t in range(PPW - NPLB, PPW):
        b, c, h = decode(t)
        pltpu.make_async_copy(plane_v.at[t % NPLB], out_hbm.at[b, c, h],
                              store_sem.at[t % NPLB]).wait()


@jax.jit
def kernel(patches, vol, offsets):
    mesh = plsc.VectorSubcoreMesh(core_axis_name="c", subcore_axis_name="s")
    run = pl.kernel(
        _sc_body,
        out_type=jax.ShapeDtypeStruct((B, C, H, H, H), jnp.float32),
        mesh=mesh,
        scratch_types=[
            pltpu.VMEM((NPLB, H, H), jnp.float32),
            pltpu.VMEM((NPAB, PR, HP), jnp.float32),
            pltpu.VMEM((BN * 3,), jnp.int32),
            pltpu.SMEM((BN * 3,), jnp.int32),
            pltpu.SMEM((NPB * 2,), jnp.int32),
            pltpu.SemaphoreType.DMA((NPLB,)),
            pltpu.SemaphoreType.DMA((NPLB,)),
            pltpu.SemaphoreType.DMA((NPAB,)),
        ],
        compiler_params=pltpu.CompilerParams(
            use_tc_tiling_on_sc=True, needs_layout_passes=False),
    )
    return run(patches, vol, offsets.reshape(-1))


# speedup vs baseline: 1.5977x; 1.1052x over previous
import functools

import jax
import jax.numpy as jnp
from jax import lax
from jax.experimental import pallas as pl
from jax.experimental.pallas import tpu as pltpu
from jax.experimental.pallas import tpu_sc as plsc

PW = 0.5
BN, C, HP = 48, 2, 64
B, H = 2, 160
NPB = BN // B
PLANES = B * C * H
NW = 32
PPW = PLANES // NW
L = 16
NPLB = 2
NPAB = 5
PR = 64
NSUB = HP // PR
RU = 4
SU = 2
KP = HP // L
KH = H // L


def _sc_body(patches_hbm, vol_hbm, off_hbm, out_hbm, plane_v, patch_v, off_t,
             off_s, wl_s, load_sem, store_sem, patch_sem):
    wid = lax.axis_index("s") * 2 + lax.axis_index("c")
    pltpu.sync_copy(off_hbm, off_t)
    lane = lax.iota(jnp.int32, L)

    def extract_body(i, carry):
        for k in range(3):
            ii = jnp.full((L,), i * 3 + k, jnp.int32)
            v = plsc.load_gather(off_t, [ii])
            off_s[i * 3 + k] = jnp.max(v)
        return carry

    lax.fori_loop(0, BN, extract_body, 0)

    def decode(t):
        p = t * NW + wid
        return p // (C * H), (p // H) % C, p % H

    def issue_load(t):
        b, c, h = decode(t)
        pltpu.async_copy(vol_hbm.at[b, c, h], plane_v.at[t % NPLB],
                         load_sem.at[t % NPLB])

    for t0 in range(NPLB - 1):
        issue_load(t0)

    def iter_body(t, carry):
        buf = lax.rem(t, NPLB)
        b, c, h = decode(t)

        def wl_body(j, m):
            i = b * NPB + j
            dh = h - off_s[i * 3]
            cond = (dh >= 0) & (dh < HP)

            @pl.when(cond)
            def _():
                wl_s[m * 2] = i
                wl_s[m * 2 + 1] = dh

            return m + cond.astype(jnp.int32)

        m = lax.fori_loop(0, NPB, wl_body, 0)

        def issue_sub(u):
            j = u // NSUB
            sub = lax.rem(u, NSUB)
            pltpu.async_copy(
                patches_hbm.at[wl_s[j * 2], c, wl_s[j * 2 + 1],
                               pl.ds(sub * PR, PR)],
                patch_v.at[lax.rem(u, NPAB)],
                patch_sem.at[lax.rem(u, NPAB)])

        nu = m * NSUB
        for u0 in range(NPAB - 1):
            @pl.when(u0 < nu)
            def _():
                issue_sub(u0)

        pltpu.make_async_copy(vol_hbm.at[b, c, h], plane_v.at[buf],
                              load_sem.at[buf]).wait()

        def sub_body(u, carry):
            pb = lax.rem(u, NPAB)
            j = u // NSUB
            sub = lax.rem(u, NSUB)
            i = wl_s[j * 2]

            @pl.when(u + NPAB - 1 < nu)
            def _():
                issue_sub(u + NPAB - 1)

            dh = wl_s[j * 2 + 1]
            pltpu.make_async_copy(
                patches_hbm.at[i, c, dh, pl.ds(sub * PR, PR)],
                patch_v.at[pb], patch_sem.at[pb]).wait()

            s1 = off_s[i * 3 + 1]
            s2 = off_s[i * 3 + 2]
            cols = tuple(lane + (s2 + k * L) for k in range(KP))
            row0 = jnp.full((L,), s1 + sub * PR, jnp.int32)

            def row_body(q, row_vec):
                r = q * RU
                xs = [patch_v[pb, r + rr, pl.ds(k * L, L)]
                      for rr in range(RU) for k in range(KP)]
                for rr in range(RU):
                    rv = row_vec + rr if rr else row_vec
                    for k in range(KP):
                        plsc.addupdate_scatter(plane_v.at[buf],
                                               [rv, cols[k]],
                                               xs[rr * KP + k])
                return row_vec + RU

            lax.fori_loop(0, PR // RU, row_body, row0)
            return carry

        lax.fori_loop(0, nu, sub_body, 0)

        def scale_body(q, cc):
            r = q * SU
            xs = [plane_v[buf, r + rr, pl.ds(k * L, L)] * PW
                  for rr in range(SU) for k in range(KH)]
            for rr in range(SU):
                for k in range(KH):
                    plane_v[buf, r + rr, pl.ds(k * L, L)] = xs[rr * KH + k]
            return cc

        lax.fori_loop(0, H // SU, scale_body, 0)
        pltpu.async_copy(plane_v.at[buf], out_hbm.at[b, c, h],
                         store_sem.at[buf])

        @pl.when(t + NPLB - 1 < PPW)
        def _():
            nbuf = lax.rem(t + NPLB - 1, NPLB)

            @pl.when(t >= 1)
            def _():
                bp, cp, hp_ = decode(t - 1)
                pltpu.make_async_copy(plane_v.at[nbuf],
                                      out_hbm.at[bp, cp, hp_],
                                      store_sem.at[nbuf]).wait()

            issue_load(t + NPLB - 1)

        return carry

    lax.fori_loop(0, PPW, iter_body, 0)

    for t in range(PPW - NPLB, PPW):
        b, c, h = decode(t)
        pltpu.make_async_copy(plane_v.at[t % NPLB], out_hbm.at[b, c, h],
                              store_sem.at[t % NPLB]).wait()


@jax.jit
def kernel(patches, vol, offsets):
    mesh = plsc.VectorSubcoreMesh(core_axis_name="c", subcore_axis_name="s")
    run = pl.kernel(
        _sc_body,
        out_type=jax.ShapeDtypeStruct((B, C, H, H, H), jnp.float32),
        mesh=mesh,
        scratch_types=[
            pltpu.VMEM((NPLB, H, H), jnp.float32),
            pltpu.VMEM((NPAB, PR, HP), jnp.float32),
            pltpu.VMEM((BN * 3,), jnp.int32),
            pltpu.SMEM((BN * 3,), jnp.int32),
            pltpu.SMEM((NPB * 2,), jnp.int32),
            pltpu.SemaphoreType.DMA((NPLB,)),
            pltpu.SemaphoreType.DMA((NPLB,)),
            pltpu.SemaphoreType.DMA((NPAB,)),
        ],
        compiler_params=pltpu.CompilerParams(
            use_tc_tiling_on_sc=True, needs_layout_passes=False),
    )
    return run(patches, vol, offsets.reshape(-1))


# speedup vs baseline: 1.8290x; 1.1448x over previous
import functools

import jax
import jax.numpy as jnp
from jax import lax
from jax.experimental import pallas as pl
from jax.experimental.pallas import tpu as pltpu
from jax.experimental.pallas import tpu_sc as plsc

PW = 0.5
BN, C, HP = 48, 2, 64
B, H = 2, 160
NPB = BN // B
PLANES = B * C * H
NW = 32
PPW = PLANES // NW
L = 16
NPLB = 2
NPAB = 5
RU = 4
SU = 2
KP = HP // L
KH = H // L


def _sc_body(patches_hbm, vol_hbm, off_hbm, out_hbm, plane_v, patch_v, off_t,
             off_s, wl_s, wl_start, load_sem, store_sem, patch_sem):
    wid = lax.axis_index("s") * 2 + lax.axis_index("c")
    pltpu.sync_copy(off_hbm, off_t)
    lane = lax.iota(jnp.int32, L)

    def extract_body(i, carry):
        for k in range(3):
            ii = jnp.full((L,), i * 3 + k, jnp.int32)
            v = plsc.load_gather(off_t, [ii])
            off_s[i * 3 + k] = jnp.max(v)
        return carry

    lax.fori_loop(0, BN, extract_body, 0)

    def decode(t):
        p = t * NW + wid
        return p // (C * H), (p // H) % C, p % H

    def build_wl(t, g):
        b, c, h = decode(t)
        wl_start[t] = g

        def wl_body(j, g):
            i = b * NPB + j
            dh = h - off_s[i * 3]
            cond = (dh >= 0) & (dh < HP)

            @pl.when(cond)
            def _():
                wl_s[g] = i | (dh << 6) | (c << 12)

            return g + cond.astype(jnp.int32)

        return lax.fori_loop(0, NPB, wl_body, g)

    ng = lax.fori_loop(0, PPW, build_wl, 0)
    wl_start[PPW] = ng

    def issue_patch(g):
        w = wl_s[g]
        i = w & 63
        dh = (w >> 6) & 63
        c = w >> 12
        pltpu.async_copy(patches_hbm.at[i, c, dh],
                         patch_v.at[lax.rem(g, NPAB)],
                         patch_sem.at[lax.rem(g, NPAB)])

    for g0 in range(NPAB - 1):
        @pl.when(g0 < ng)
        def _():
            issue_patch(g0)

    def issue_load(t):
        b, c, h = decode(t)
        pltpu.async_copy(vol_hbm.at[b, c, h], plane_v.at[t % NPLB],
                         load_sem.at[t % NPLB])

    for t0 in range(NPLB - 1):
        issue_load(t0)

    def iter_body(t, carry):
        buf = lax.rem(t, NPLB)
        b, c, h = decode(t)
        g_end = wl_start[t + 1]

        pltpu.make_async_copy(vol_hbm.at[b, c, h], plane_v.at[buf],
                              load_sem.at[buf]).wait()

        def patch_body(u, carry):
            pb = lax.rem(u, NPAB)
            w = wl_s[u]
            i = w & 63
            dh = (w >> 6) & 63

            @pl.when(u + NPAB - 1 < ng)
            def _():
                issue_patch(u + NPAB - 1)

            pltpu.make_async_copy(patches_hbm.at[i, c, dh], patch_v.at[pb],
                                  patch_sem.at[pb]).wait()

            s1 = off_s[i * 3 + 1]
            s2 = off_s[i * 3 + 2]
            cols = tuple(lane + (s2 + k * L) for k in range(KP))
            row0 = jnp.full((L,), s1, jnp.int32)

            def row_body(q, row_vec):
                r = q * RU
                xs = [patch_v[pb, r + rr, pl.ds(k * L, L)]
                      for rr in range(RU) for k in range(KP)]
                for rr in range(RU):
                    rv = row_vec + rr if rr else row_vec
                    for k in range(KP):
                        plsc.addupdate_scatter(plane_v.at[buf],
                                               [rv, cols[k]],
                                               xs[rr * KP + k])
                return row_vec + RU

            lax.fori_loop(0, HP // RU, row_body, row0)
            return carry

        lax.fori_loop(wl_start[t], g_end, patch_body, 0)

        @pl.when(t + NPLB - 1 < PPW)
        def _():
            nbuf = lax.rem(t + NPLB - 1, NPLB)

            @pl.when(t >= 1)
            def _():
                bp, cp, hp_ = decode(t - 1)
                pltpu.make_async_copy(plane_v.at[nbuf],
                                      out_hbm.at[bp, cp, hp_],
                                      store_sem.at[nbuf]).wait()

            issue_load(t + NPLB - 1)

        def scale_body(q, cc):
            r = q * SU
            xs = [plane_v[buf, r + rr, pl.ds(k * L, L)] * PW
                  for rr in range(SU) for k in range(KH)]
            for rr in range(SU):
                for k in range(KH):
                    plane_v[buf, r + rr, pl.ds(k * L, L)] = xs[rr * KH + k]
            return cc

        lax.fori_loop(0, H // SU, scale_body, 0)
        pltpu.async_copy(plane_v.at[buf], out_hbm.at[b, c, h],
                         store_sem.at[buf])
        return carry

    lax.fori_loop(0, PPW, iter_body, 0)

    for t in range(PPW - NPLB, PPW):
        b, c, h = decode(t)
        pltpu.make_async_copy(plane_v.at[t % NPLB], out_hbm.at[b, c, h],
                              store_sem.at[t % NPLB]).wait()


@jax.jit
def kernel(patches, vol, offsets):
    mesh = plsc.VectorSubcoreMesh(core_axis_name="c", subcore_axis_name="s")
    run = pl.kernel(
        _sc_body,
        out_type=jax.ShapeDtypeStruct((B, C, H, H, H), jnp.float32),
        mesh=mesh,
        scratch_types=[
            pltpu.VMEM((NPLB, H, H), jnp.float32),
            pltpu.VMEM((NPAB, HP, HP), jnp.float32),
            pltpu.VMEM((BN * 3,), jnp.int32),
            pltpu.SMEM((BN * 3,), jnp.int32),
            pltpu.SMEM((NPB * PPW,), jnp.int32),
            pltpu.SMEM((PPW + 1,), jnp.int32),
            pltpu.SemaphoreType.DMA((NPLB,)),
            pltpu.SemaphoreType.DMA((NPLB,)),
            pltpu.SemaphoreType.DMA((NPAB,)),
        ],
        compiler_params=pltpu.CompilerParams(
            use_tc_tiling_on_sc=True, needs_layout_passes=False),
    )
    return run(patches, vol, offsets.reshape(-1))
